# Initial kernel scaffold; baseline (speedup 1.0000x reference)
import jax
import jax.numpy as jnp
from jax.experimental import pallas as pl


def kernel(x, edge_index, Wl1, bl1, Wr1, Wl2, bl2, Wr2):
    raise NotImplementedError("write your pallas kernel here")


# speedup vs baseline: 6.4660x; 6.4660x over previous
import functools

import jax
import jax.numpy as jnp
from jax import lax
from jax.experimental import pallas as pl
from jax.experimental.pallas import tpu as pltpu
from jax.experimental.pallas import tpu_sc as plsc

N_NODES = 10000
N_EDGES = 320000
D = 128

NUM_CORES = 2
NUM_SUBCORES = 16
NW = NUM_CORES * NUM_SUBCORES
EDGES_PER_CORE = N_EDGES // NUM_CORES
EDGES_PER_TILE = N_EDGES // NW
CHUNK = 80
NCHUNK = EDGES_PER_TILE // CHUNK
OUT_TILES = 10
ROWS_PER_TILE = N_NODES // OUT_TILES
ZROWS = 200
CNT_CHUNK = 2000


def _sc_agg_body(with_counts, *refs):
  if with_counts:
    (x_hbm, src_hbm, dst_hbm, out_p, out_c,
     src_all, dst_v, rows_v, zbuf, ones_v, zcnt, acc, cnt) = refs
  else:
    (x_hbm, src_hbm, dst_hbm, out_p,
     src_all, dst_v, rows_v, zbuf, acc) = refs

  cid = lax.axis_index("c")
  sid = lax.axis_index("s")

  zeros16 = jnp.zeros((16,), jnp.float32)

  def _zrow(i, _):
    for j in range(D // 16):
      zbuf[i, pl.ds(j * 16, 16)] = zeros16
    return 0

  lax.fori_loop(0, ZROWS, _zrow, 0)

  @pl.when(sid < OUT_TILES)
  def _():
    for r in range(ROWS_PER_TILE // ZROWS):
      pltpu.sync_copy(
          zbuf, acc.at[pl.ds(sid * ROWS_PER_TILE + r * ZROWS, ZROWS), :])

  if with_counts:
    ones16 = jnp.ones((16,), jnp.float32)
    for j in range(CHUNK // 16):
      ones_v[pl.ds(j * 16, 16)] = ones16

    def _zc(i, _):
      zcnt[pl.ds(i * 16, 16)] = zeros16
      return 0

    lax.fori_loop(0, CNT_CHUNK // 16, _zc, 0)

    @pl.when(sid < N_NODES // CNT_CHUNK)
    def _():
      pltpu.sync_copy(zcnt, cnt.at[pl.ds(sid * CNT_CHUNK, CNT_CHUNK)])

  plsc.subcore_barrier()

  ebase = (cid * NUM_SUBCORES + sid) * EDGES_PER_TILE
  pltpu.sync_copy(src_hbm.at[pl.ds(ebase, EDGES_PER_TILE)], src_all)

  def _chunk(g, _):
    off = g * CHUNK
    pltpu.sync_copy(dst_hbm.at[pl.ds(ebase + off, CHUNK)], dst_v)
    pltpu.sync_copy(x_hbm.at[src_all.at[pl.ds(off, CHUNK)]], rows_v)
    pltpu.sync_copy(rows_v, acc.at[dst_v], add=True)
    if with_counts:
      pltpu.sync_copy(ones_v, cnt.at[dst_v], add=True)
    return 0

  lax.fori_loop(0, NCHUNK, _chunk, 0)

  plsc.subcore_barrier()

  @pl.when(sid < OUT_TILES)
  def _():
    pltpu.sync_copy(
        acc.at[pl.ds(sid * ROWS_PER_TILE, ROWS_PER_TILE), :],
        out_p.at[cid, pl.ds(sid * ROWS_PER_TILE, ROWS_PER_TILE), :],
    )
  if with_counts:
    @pl.when(sid < N_NODES // CNT_CHUNK)
    def _():
      pltpu.sync_copy(cnt.at[pl.ds(sid * CNT_CHUNK, CNT_CHUNK)], zcnt)
      pltpu.sync_copy(
          zcnt, out_c.at[pl.ds(cid * N_NODES + sid * CNT_CHUNK, CNT_CHUNK)])


def _make_sc_agg(with_counts):
  mesh = plsc.VectorSubcoreMesh(
      core_axis_name="c", subcore_axis_name="s",
      num_cores=NUM_CORES, num_subcores=NUM_SUBCORES,
  )
  out_type = [jax.ShapeDtypeStruct((NUM_CORES, N_NODES, D), jnp.float32)]
  if with_counts:
    out_type.append(jax.ShapeDtypeStruct((NUM_CORES * N_NODES,), jnp.float32))
  scratch = [
      pltpu.VMEM((EDGES_PER_TILE,), jnp.int32),
      pltpu.VMEM((CHUNK,), jnp.int32),
      pltpu.VMEM((CHUNK, D), jnp.float32),
      pltpu.VMEM((ZROWS, D), jnp.float32),
  ]
  if with_counts:
    scratch += [
        pltpu.VMEM((CHUNK,), jnp.float32),
        pltpu.VMEM((CNT_CHUNK,), jnp.float32),
    ]
  scratch.append(pltpu.VMEM_SHARED((N_NODES, D), jnp.float32))
  if with_counts:
    scratch.append(pltpu.VMEM_SHARED((N_NODES,), jnp.float32))

  return pl.kernel(
      functools.partial(_sc_agg_body, with_counts),
      out_type=tuple(out_type) if with_counts else out_type[0],
      mesh=mesh,
      scratch_types=scratch,
  )


_sc_agg_with_counts = _make_sc_agg(True)
_sc_agg_no_counts = _make_sc_agg(False)


def _tc_layer_body(relu, p_ref, c_ref, x_ref, wlt_ref, wrt_ref, bl_ref, o_ref):
  c = c_ref[0] + c_ref[1]
  inv = 1.0 / jnp.maximum(c, 1.0)
  mean = (p_ref[0] + p_ref[1]) * inv
  acc = jnp.dot(mean, wlt_ref[...], preferred_element_type=jnp.float32)
  acc = acc + jnp.dot(x_ref[...], wrt_ref[...], preferred_element_type=jnp.float32)
  acc = acc + bl_ref[...]
  if relu:
    acc = jnp.maximum(acc, 0.0)
  o_ref[...] = acc


def _make_tc_layer(relu, block=1000):
  nblk = N_NODES // block
  return pl.pallas_call(
      functools.partial(_tc_layer_body, relu),
      grid=(nblk,),
      in_specs=[
          pl.BlockSpec((NUM_CORES, block, D), lambda i: (0, i, 0)),
          pl.BlockSpec((NUM_CORES, block, 1), lambda i: (0, i, 0)),
          pl.BlockSpec((block, D), lambda i: (i, 0)),
          pl.BlockSpec((D, D), lambda i: (0, 0)),
          pl.BlockSpec((D, D), lambda i: (0, 0)),
          pl.BlockSpec((1, D), lambda i: (0, 0)),
      ],
      out_specs=pl.BlockSpec((block, D), lambda i: (i, 0)),
      out_shape=jax.ShapeDtypeStruct((N_NODES, D), jnp.float32),
  )


_tc_layer_relu = _make_tc_layer(True)
_tc_layer_lin = _make_tc_layer(False)


def kernel(x, edge_index, Wl1, bl1, Wr1, Wl2, bl2, Wr2):
  src = edge_index[0].astype(jnp.int32)
  dst = edge_index[1].astype(jnp.int32)

  p1, cnt = _sc_agg_with_counts(x, src, dst)
  cnt3 = cnt.reshape(NUM_CORES, N_NODES, 1)
  h = _tc_layer_relu(p1, cnt3, x, Wl1.T, Wr1.T, bl1.reshape(1, D))
  p2 = _sc_agg_no_counts(h, src, dst)
  out = _tc_layer_lin(p2, cnt3, h, Wl2.T, Wr2.T, bl2.reshape(1, D))
  return out


# speedup vs baseline: 9.9445x; 1.5380x over previous
import functools

import jax
import jax.numpy as jnp
from jax import lax
from jax.experimental import pallas as pl
from jax.experimental.pallas import tpu as pltpu
from jax.experimental.pallas import tpu_sc as plsc

N_NODES = 10000
N_EDGES = 320000
D = 128

NUM_CORES = 2
NUM_SUBCORES = 16
NW = NUM_CORES * NUM_SUBCORES
EDGES_PER_TILE = N_EDGES // NW
CHUNK = 80
NCHUNK = EDGES_PER_TILE // CHUNK
OUT_TILES = 10
ROWS_PER_TILE = N_NODES // OUT_TILES
CNT_CHUNK = 2000


def _sc_agg_body(with_counts, *refs):
  if with_counts:
    (x_hbm, src_hbm, dst3_hbm, out_p, out_c,
     src_all, dst_all, rows0, rows1, ones_v, zcnt, acc, cnt,
     gsem0, gsem1, ssem0, ssem1, csem0, csem1) = refs
  else:
    (x_hbm, src_hbm, dst3_hbm, out_p,
     src_all, dst_all, rows0, rows1, acc,
     gsem0, gsem1, ssem0, ssem1) = refs
    ones_v = zcnt = cnt = csem0 = csem1 = None

  cid = lax.axis_index("c")
  sid = lax.axis_index("s")
  wid = cid * NUM_SUBCORES + sid

  zeros16 = jnp.zeros((16,), jnp.float32)

  def _zrow(i, _):
    for j in range(D // 16):
      rows0[i, pl.ds(j * 16, 16)] = zeros16
    return 0

  lax.fori_loop(0, CHUNK, _zrow, 0)

  @pl.when(sid < OUT_TILES)
  def _():
    for r in range(ROWS_PER_TILE // CHUNK):
      pltpu.sync_copy(
          rows0, acc.at[pl.ds(sid * ROWS_PER_TILE + r * CHUNK, CHUNK), :])
    pltpu.sync_copy(
        rows0.at[pl.ds(0, ROWS_PER_TILE % CHUNK), :],
        acc.at[pl.ds(sid * ROWS_PER_TILE + ROWS_PER_TILE - ROWS_PER_TILE % CHUNK,
                     ROWS_PER_TILE % CHUNK), :])

  if with_counts:
    ones16 = jnp.ones((16,), jnp.float32)
    for j in range(CHUNK // 16):
      ones_v[pl.ds(j * 16, 16)] = ones16

    def _zc(i, _):
      zcnt[pl.ds(i * 16, 16)] = zeros16
      return 0

    lax.fori_loop(0, CNT_CHUNK // 16, _zc, 0)

    @pl.when(sid < N_NODES // CNT_CHUNK)
    def _():
      pltpu.sync_copy(zcnt, cnt.at[pl.ds(sid * CNT_CHUNK, CNT_CHUNK)])

  plsc.subcore_barrier()

  ebase = wid * EDGES_PER_TILE
  pltpu.sync_copy(src_hbm.at[pl.ds(ebase, EDGES_PER_TILE)], src_all)
  pltpu.sync_copy(dst3_hbm.at[wid], dst_all)

  rows = (rows0, rows1)
  gsem = (gsem0, gsem1)
  ssem = (ssem0, ssem1)
  csem = (csem0, csem1)

  def gat(g, b):
    pltpu.async_copy(
        x_hbm.at[src_all.at[pl.ds(g * CHUNK, CHUNK)]], rows[b], gsem[b])

  def gat_wait(b):
    pltpu.make_async_copy(
        x_hbm.at[src_all.at[pl.ds(0, CHUNK)]], rows[b], gsem[b]).wait()

  def scat(g, b):
    pltpu.async_copy(rows[b], acc.at[dst_all.at[g]], ssem[b], add=True)
    if with_counts:
      pltpu.async_copy(ones_v, cnt.at[dst_all.at[g]], csem[b], add=True)

  def scat_wait(g, b):
    pltpu.make_async_copy(rows[b], acc.at[dst_all.at[g]], ssem[b]).wait()
    if with_counts:
      pltpu.make_async_copy(ones_v, cnt.at[dst_all.at[g]], csem[b]).wait()

  gat(0, 0)
  gat_wait(0)
  scat(0, 0)
  gat(1, 1)

  def _pair(k, _):
    g = 2 * k + 1
    gat_wait(1)
    scat(g, 1)
    scat_wait(g - 1, 0)
    gat(g + 1, 0)
    g = 2 * k + 2
    gat_wait(0)
    scat(g, 0)
    scat_wait(g - 1, 1)

    @pl.when(g + 1 < NCHUNK)
    def _():
      gat(g + 1, 1)

    return 0

  lax.fori_loop(0, (NCHUNK - 1) // 2, _pair, 0)
  scat_wait(NCHUNK - 1, 0)

  plsc.subcore_barrier()

  @pl.when(sid < OUT_TILES)
  def _():
    pltpu.sync_copy(
        acc.at[pl.ds(sid * ROWS_PER_TILE, ROWS_PER_TILE), :],
        out_p.at[cid, pl.ds(sid * ROWS_PER_TILE, ROWS_PER_TILE), :],
    )
  if with_counts:
    @pl.when(sid < N_NODES // CNT_CHUNK)
    def _():
      pltpu.sync_copy(cnt.at[pl.ds(sid * CNT_CHUNK, CNT_CHUNK)], zcnt)
      pltpu.sync_copy(
          zcnt, out_c.at[pl.ds(cid * N_NODES + sid * CNT_CHUNK, CNT_CHUNK)])


def _make_sc_agg(with_counts):
  mesh = plsc.VectorSubcoreMesh(
      core_axis_name="c", subcore_axis_name="s",
      num_cores=NUM_CORES, num_subcores=NUM_SUBCORES,
  )
  out_type = [jax.ShapeDtypeStruct((NUM_CORES, N_NODES, D), jnp.float32)]
  if with_counts:
    out_type.append(jax.ShapeDtypeStruct((NUM_CORES * N_NODES,), jnp.float32))
  scratch = [
      pltpu.VMEM((EDGES_PER_TILE,), jnp.int32),
      pltpu.VMEM((NCHUNK, CHUNK), jnp.int32),
      pltpu.VMEM((CHUNK, D), jnp.float32),
      pltpu.VMEM((CHUNK, D), jnp.float32),
  ]
  if with_counts:
    scratch += [
        pltpu.VMEM((CHUNK,), jnp.float32),
        pltpu.VMEM((CNT_CHUNK,), jnp.float32),
    ]
  scratch.append(pltpu.VMEM_SHARED((N_NODES, D), jnp.float32))
  if with_counts:
    scratch.append(pltpu.VMEM_SHARED((N_NODES,), jnp.float32))
  nsem = 6 if with_counts else 4
  scratch += [pltpu.SemaphoreType.DMA] * nsem

  return pl.kernel(
      functools.partial(_sc_agg_body, with_counts),
      out_type=tuple(out_type) if with_counts else out_type[0],
      mesh=mesh,
      scratch_types=scratch,
  )


_sc_agg_with_counts = _make_sc_agg(True)
_sc_agg_no_counts = _make_sc_agg(False)


def _tc_layer_body(relu, p_ref, c_ref, x_ref, wlt_ref, wrt_ref, bl_ref, o_ref):
  c = c_ref[0] + c_ref[1]
  inv = 1.0 / jnp.maximum(c, 1.0)
  mean = (p_ref[0] + p_ref[1]) * inv
  acc = jnp.dot(mean, wlt_ref[...], preferred_element_type=jnp.float32)
  acc = acc + jnp.dot(x_ref[...], wrt_ref[...], preferred_element_type=jnp.float32)
  acc = acc + bl_ref[...]
  if relu:
    acc = jnp.maximum(acc, 0.0)
  o_ref[...] = acc


def _make_tc_layer(relu, block=1000):
  nblk = N_NODES // block
  return pl.pallas_call(
      functools.partial(_tc_layer_body, relu),
      grid=(nblk,),
      in_specs=[
          pl.BlockSpec((NUM_CORES, block, D), lambda i: (0, i, 0)),
          pl.BlockSpec((NUM_CORES, block, 1), lambda i: (0, i, 0)),
          pl.BlockSpec((block, D), lambda i: (i, 0)),
          pl.BlockSpec((D, D), lambda i: (0, 0)),
          pl.BlockSpec((D, D), lambda i: (0, 0)),
          pl.BlockSpec((1, D), lambda i: (0, 0)),
      ],
      out_specs=pl.BlockSpec((block, D), lambda i: (i, 0)),
      out_shape=jax.ShapeDtypeStruct((N_NODES, D), jnp.float32),
  )


_tc_layer_relu = _make_tc_layer(True)
_tc_layer_lin = _make_tc_layer(False)


def kernel(x, edge_index, Wl1, bl1, Wr1, Wl2, bl2, Wr2):
  src = edge_index[0].astype(jnp.int32)
  dst = edge_index[1].astype(jnp.int32)
  dst3 = dst.reshape(NW, NCHUNK, CHUNK)

  p1, cnt = _sc_agg_with_counts(x, src, dst3)
  cnt3 = cnt.reshape(NUM_CORES, N_NODES, 1)
  h = _tc_layer_relu(p1, cnt3, x, Wl1.T, Wr1.T, bl1.reshape(1, D))
  p2 = _sc_agg_no_counts(h, src, dst3)
  out = _tc_layer_lin(p2, cnt3, h, Wl2.T, Wr2.T, bl2.reshape(1, D))
  return out


# speedup vs baseline: 14.0129x; 1.4091x over previous
import functools

import jax
import jax.numpy as jnp
from jax import lax
from jax.experimental import pallas as pl
from jax.experimental.pallas import tpu as pltpu
from jax.experimental.pallas import tpu_sc as plsc

N_NODES = 10000
N_EDGES = 320000
D = 128

NUM_CORES = 2
NUM_SUBCORES = 16
NW = NUM_CORES * NUM_SUBCORES
EDGES_PER_TILE = N_EDGES // NW
CHUNK = 80
NCHUNK = EDGES_PER_TILE // CHUNK
OUT_TILES = 10
ROWS_PER_TILE = N_NODES // OUT_TILES
CNT_CHUNK = 200


def _sc_agg_body(with_counts, *refs):
  if with_counts:
    (x_hbm, src_hbm, dst3_hbm, out_p, out_c,
     sv0, sv1, sv2, dst_all, rows0, rows1, rows2, ones_v, zcnt, acc, cnt,
     dsem0, dsem1, dsem2, gsem0, gsem1, gsem2,
     ssem0, ssem1, ssem2, csem0, csem1, csem2) = refs
  else:
    (x_hbm, src_hbm, dst3_hbm, out_p,
     sv0, sv1, sv2, dst_all, rows0, rows1, rows2, acc,
     dsem0, dsem1, dsem2, gsem0, gsem1, gsem2,
     ssem0, ssem1, ssem2) = refs
    ones_v = zcnt = cnt = csem0 = csem1 = csem2 = None

  cid = lax.axis_index("c")
  sid = lax.axis_index("s")
  wid = cid * NUM_SUBCORES + sid

  zeros16 = jnp.zeros((16,), jnp.float32)

  def _zrow(i, _):
    for j in range(D // 16):
      rows0[i, pl.ds(j * 16, 16)] = zeros16
    return 0

  lax.fori_loop(0, CHUNK, _zrow, 0)

  @pl.when(sid < OUT_TILES)
  def _():
    for r in range(ROWS_PER_TILE // CHUNK):
      pltpu.sync_copy(
          rows0, acc.at[pl.ds(sid * ROWS_PER_TILE + r * CHUNK, CHUNK), :])
    pltpu.sync_copy(
        rows0.at[pl.ds(0, ROWS_PER_TILE % CHUNK), :],
        acc.at[pl.ds(sid * ROWS_PER_TILE + ROWS_PER_TILE - ROWS_PER_TILE % CHUNK,
                     ROWS_PER_TILE % CHUNK), :])

  if with_counts:
    ones16 = jnp.ones((16,), jnp.float32)
    for j in range(CHUNK // 16):
      ones_v[pl.ds(j * 16, 16)] = ones16

    def _zc(i, _):
      zcnt[pl.ds(i * 16, 16)] = zeros16
      return 0

    lax.fori_loop(0, (CNT_CHUNK + 15) // 16, _zc, 0)

    @pl.when(sid < OUT_TILES)
    def _():
      for r in range(ROWS_PER_TILE // CNT_CHUNK):
        pltpu.sync_copy(
            zcnt.at[pl.ds(0, CNT_CHUNK)],
            cnt.at[pl.ds(sid * ROWS_PER_TILE + r * CNT_CHUNK, CNT_CHUNK)])

  plsc.subcore_barrier()

  ebase = wid * EDGES_PER_TILE
  pltpu.sync_copy(dst3_hbm.at[wid], dst_all)

  sv = (sv0, sv1, sv2)
  rows = (rows0, rows1, rows2)
  dsem = (dsem0, dsem1, dsem2)
  gsem = (gsem0, gsem1, gsem2)
  ssem = (ssem0, ssem1, ssem2)
  csem = (csem0, csem1, csem2)

  def ld_src(g, b):
    pltpu.async_copy(
        src_hbm.at[pl.ds(ebase + g * CHUNK, CHUNK)], sv[b], dsem[b])

  def ld_src_wait(b):
    pltpu.make_async_copy(src_hbm.at[pl.ds(0, CHUNK)], sv[b], dsem[b]).wait()

  def gat(b):
    pltpu.async_copy(x_hbm.at[sv[b]], rows[b], gsem[b])

  def gat_wait(b):
    pltpu.make_async_copy(x_hbm.at[sv[b]], rows[b], gsem[b]).wait()

  def scat(g, b):
    pltpu.async_copy(rows[b], acc.at[dst_all.at[g]], ssem[b], add=True)
    if with_counts:
      pltpu.async_copy(ones_v, cnt.at[dst_all.at[g]], csem[b], add=True)

  def scat_wait(g, b):
    pltpu.make_async_copy(rows[b], acc.at[dst_all.at[g]], ssem[b]).wait()
    if with_counts:
      pltpu.make_async_copy(ones_v, cnt.at[dst_all.at[g]], csem[b]).wait()

  def step(g, b, first=False):
    gat_wait(b)
    scat(g, b)

    @pl.when(g + 3 < NCHUNK)
    def _():
      ld_src(g + 3, b)
    if not first:
      scat_wait(g - 1, (b + 2) % 3)

    @pl.when(g + 2 < NCHUNK)
    def _():
      ld_src_wait((b + 2) % 3)
      gat((b + 2) % 3)

  ld_src(0, 0)
  ld_src(1, 1)
  ld_src(2, 2)
  ld_src_wait(0)
  gat(0)
  ld_src_wait(1)
  gat(1)
  step(0, 0, first=True)
  step(1, 1)

  def _trip(k, _):
    g = 3 * k + 2
    step(g, 2)
    step(g + 1, 0)
    step(g + 2, 1)
    return 0

  lax.fori_loop(0, (NCHUNK - 2) // 3, _trip, 0)
  scat_wait(NCHUNK - 1, (NCHUNK - 1) % 3)

  plsc.subcore_barrier()

  @pl.when(sid < OUT_TILES)
  def _():
    pltpu.sync_copy(
        acc.at[pl.ds(sid * ROWS_PER_TILE, ROWS_PER_TILE), :],
        out_p.at[cid, pl.ds(sid * ROWS_PER_TILE, ROWS_PER_TILE), :],
    )
  if with_counts:
    @pl.when(sid < OUT_TILES)
    def _():
      for r in range(ROWS_PER_TILE // CNT_CHUNK):
        off = sid * ROWS_PER_TILE + r * CNT_CHUNK
        pltpu.sync_copy(cnt.at[pl.ds(off, CNT_CHUNK)],
                        zcnt.at[pl.ds(0, CNT_CHUNK)])
        pltpu.sync_copy(zcnt.at[pl.ds(0, CNT_CHUNK)],
                        out_c.at[pl.ds(cid * N_NODES + off, CNT_CHUNK)])


def _make_sc_agg(with_counts):
  mesh = plsc.VectorSubcoreMesh(
      core_axis_name="c", subcore_axis_name="s",
      num_cores=NUM_CORES, num_subcores=NUM_SUBCORES,
  )
  out_type = [jax.ShapeDtypeStruct((NUM_CORES, N_NODES, D), jnp.float32)]
  if with_counts:
    out_type.append(jax.ShapeDtypeStruct((NUM_CORES * N_NODES,), jnp.float32))
  scratch = [
      pltpu.VMEM((CHUNK,), jnp.int32),
      pltpu.VMEM((CHUNK,), jnp.int32),
      pltpu.VMEM((CHUNK,), jnp.int32),
      pltpu.VMEM((NCHUNK, CHUNK), jnp.int32),
      pltpu.VMEM((CHUNK, D), jnp.float32),
      pltpu.VMEM((CHUNK, D), jnp.float32),
      pltpu.VMEM((CHUNK, D), jnp.float32),
  ]
  if with_counts:
    scratch += [
        pltpu.VMEM((CHUNK,), jnp.float32),
        pltpu.VMEM((16 * ((CNT_CHUNK + 15) // 16),), jnp.float32),
    ]
  scratch.append(pltpu.VMEM_SHARED((N_NODES, D), jnp.float32))
  if with_counts:
    scratch.append(pltpu.VMEM_SHARED((N_NODES,), jnp.float32))
  nsem = 12 if with_counts else 9
  scratch += [pltpu.SemaphoreType.DMA] * nsem

  return pl.kernel(
      functools.partial(_sc_agg_body, with_counts),
      out_type=tuple(out_type) if with_counts else out_type[0],
      mesh=mesh,
      scratch_types=scratch,
  )


_sc_agg_with_counts = _make_sc_agg(True)
_sc_agg_no_counts = _make_sc_agg(False)


def _tc_layer_body(relu, p_ref, c_ref, x_ref, wlt_ref, wrt_ref, bl_ref, o_ref):
  c = c_ref[0] + c_ref[1]
  inv = 1.0 / jnp.maximum(c, 1.0)
  mean = (p_ref[0] + p_ref[1]) * inv
  acc = jnp.dot(mean, wlt_ref[...], preferred_element_type=jnp.float32)
  acc = acc + jnp.dot(x_ref[...], wrt_ref[...], preferred_element_type=jnp.float32)
  acc = acc + bl_ref[...]
  if relu:
    acc = jnp.maximum(acc, 0.0)
  o_ref[...] = acc


def _make_tc_layer(relu, block=1000):
  nblk = N_NODES // block
  return pl.pallas_call(
      functools.partial(_tc_layer_body, relu),
      grid=(nblk,),
      in_specs=[
          pl.BlockSpec((NUM_CORES, block, D), lambda i: (0, i, 0)),
          pl.BlockSpec((NUM_CORES, block, 1), lambda i: (0, i, 0)),
          pl.BlockSpec((block, D), lambda i: (i, 0)),
          pl.BlockSpec((D, D), lambda i: (0, 0)),
          pl.BlockSpec((D, D), lambda i: (0, 0)),
          pl.BlockSpec((1, D), lambda i: (0, 0)),
      ],
      out_specs=pl.BlockSpec((block, D), lambda i: (i, 0)),
      out_shape=jax.ShapeDtypeStruct((N_NODES, D), jnp.float32),
  )


_tc_layer_relu = _make_tc_layer(True)
_tc_layer_lin = _make_tc_layer(False)


def kernel(x, edge_index, Wl1, bl1, Wr1, Wl2, bl2, Wr2):
  src = edge_index[0].astype(jnp.int32)
  dst = edge_index[1].astype(jnp.int32)
  dst3 = dst.reshape(NW, NCHUNK, CHUNK)

  p1, cnt = _sc_agg_with_counts(x, src, dst3)
  cnt3 = cnt.reshape(NUM_CORES, N_NODES, 1)
  h = _tc_layer_relu(p1, cnt3, x, Wl1.T, Wr1.T, bl1.reshape(1, D))
  p2 = _sc_agg_no_counts(h, src, dst3)
  out = _tc_layer_lin(p2, cnt3, h, Wl2.T, Wr2.T, bl2.reshape(1, D))
  return out


# speedup vs baseline: 15.0502x; 1.0740x over previous
import functools

import jax
import jax.numpy as jnp
from jax import lax
from jax.experimental import pallas as pl
from jax.experimental.pallas import tpu as pltpu
from jax.experimental.pallas import tpu_sc as plsc

N_NODES = 10000
N_EDGES = 320000
D = 128

NUM_CORES = 2
NUM_SUBCORES = 16
NW = NUM_CORES * NUM_SUBCORES
EDGES_PER_TILE = N_EDGES // NW
CHUNK = 80
NCHUNK = EDGES_PER_TILE // CHUNK
NBUF = 4
NSRC = 8
PRE = 5
OUT_TILES = 10
ROWS_PER_TILE = N_NODES // OUT_TILES
CNT_CHUNK = 200
CNT_PAD = 16 * ((CNT_CHUNK + 15) // 16)


def _sc_agg_body(with_counts, *refs):
  nsem = NSRC + 3 * NBUF + (NBUF if with_counts else 0)
  if with_counts:
    (x_hbm, ei_hbm, out_p, out_c, ones_v, zcnt, acc, cnt) = refs[:8]
    sv = refs[8:8 + NSRC]
    dv = refs[8 + NSRC:8 + NSRC + NBUF]
    rows = refs[8 + NSRC + NBUF:8 + NSRC + 2 * NBUF]
    sems = refs[8 + NSRC + 2 * NBUF:]
  else:
    (x_hbm, ei_hbm, out_p, acc) = refs[:4]
    ones_v = zcnt = cnt = None
    sv = refs[4:4 + NSRC]
    dv = refs[4 + NSRC:4 + NSRC + NBUF]
    rows = refs[4 + NSRC + NBUF:4 + NSRC + 2 * NBUF]
    sems = refs[4 + NSRC + 2 * NBUF:]
  isem = sems[:NSRC]
  jsem = sems[NSRC:NSRC + NBUF]
  gsem = sems[NSRC + NBUF:NSRC + 2 * NBUF]
  ssem = sems[NSRC + 2 * NBUF:NSRC + 3 * NBUF]
  csem = sems[NSRC + 3 * NBUF:] if with_counts else None

  cid = lax.axis_index("c")
  sid = lax.axis_index("s")
  wid = cid * NUM_SUBCORES + sid

  zeros16 = jnp.zeros((16,), jnp.float32)

  def _zrow(i, _):
    for j in range(D // 16):
      rows[0][i, pl.ds(j * 16, 16)] = zeros16
    return 0

  lax.fori_loop(0, CHUNK, _zrow, 0)

  @pl.when(sid < OUT_TILES)
  def _():
    for r in range(ROWS_PER_TILE // CHUNK):
      pltpu.sync_copy(
          rows[0], acc.at[pl.ds(sid * ROWS_PER_TILE + r * CHUNK, CHUNK), :])
    pltpu.sync_copy(
        rows[0].at[pl.ds(0, ROWS_PER_TILE % CHUNK), :],
        acc.at[pl.ds(sid * ROWS_PER_TILE + ROWS_PER_TILE - ROWS_PER_TILE % CHUNK,
                     ROWS_PER_TILE % CHUNK), :])

  if with_counts:
    ones16 = jnp.ones((16,), jnp.float32)
    for j in range(CHUNK // 16):
      ones_v[pl.ds(j * 16, 16)] = ones16

    def _zc(i, _):
      zcnt[pl.ds(i * 16, 16)] = zeros16
      return 0

    lax.fori_loop(0, CNT_PAD // 16, _zc, 0)

    @pl.when(sid < OUT_TILES)
    def _():
      for r in range(ROWS_PER_TILE // CNT_CHUNK):
        pltpu.sync_copy(
            zcnt.at[pl.ds(0, CNT_CHUNK)],
            cnt.at[pl.ds(sid * ROWS_PER_TILE + r * CNT_CHUNK, CNT_CHUNK)])

  plsc.subcore_barrier()

  ebase = wid * EDGES_PER_TILE

  def ld_src(g, m):
    pltpu.async_copy(
        ei_hbm.at[pl.ds(ebase + g * CHUNK, CHUNK)], sv[m % NSRC],
        isem[m % NSRC])

  def src_wait(m):
    pltpu.make_async_copy(
        ei_hbm.at[pl.ds(0, CHUNK)], sv[m % NSRC], isem[m % NSRC]).wait()

  def ld_dst(g, m):
    pltpu.async_copy(
        ei_hbm.at[pl.ds(N_EDGES + ebase + g * CHUNK, CHUNK)], dv[m % NBUF],
        jsem[m % NBUF])

  def dst_wait(m):
    pltpu.make_async_copy(
        ei_hbm.at[pl.ds(0, CHUNK)], dv[m % NBUF], jsem[m % NBUF]).wait()

  def gat(m):
    pltpu.async_copy(x_hbm.at[sv[m % NSRC]], rows[m % NBUF], gsem[m % NBUF])

  def gat_wait(m):
    pltpu.make_async_copy(
        x_hbm.at[sv[m % NSRC]], rows[m % NBUF], gsem[m % NBUF]).wait()

  def scat(m):
    pltpu.async_copy(rows[m % NBUF], acc.at[dv[m % NBUF]], ssem[m % NBUF],
                     add=True)
    if with_counts:
      pltpu.async_copy(ones_v, cnt.at[dv[m % NBUF]], csem[m % NBUF], add=True)

  def scat_wait(m):
    pltpu.make_async_copy(
        rows[m % NBUF], acc.at[dv[m % NBUF]], ssem[m % NBUF]).wait()
    if with_counts:
      pltpu.make_async_copy(
          ones_v, cnt.at[dv[m % NBUF]], csem[m % NBUF]).wait()

  def step(g, m, first=False):
    gat_wait(m)
    dst_wait(m)
    scat(m)
    if not first:
      scat_wait(m - 1)

    @pl.when(g + 4 < NCHUNK)
    def _():
      ld_src(g + 4, m + 4)

    @pl.when(g + 3 < NCHUNK)
    def _():
      ld_dst(g + 3, m + 3)
      src_wait(m + 3)
      gat(m + 3)

  for g in range(4):
    ld_src(g, g)
  for g in range(3):
    ld_dst(g, g)
  for g in range(3):
    src_wait(g)
    gat(g)
  step(0, 0, first=True)
  for g in range(1, PRE):
    step(g, g)

  def _oct(k, _):
    for j in range(8):
      step(8 * k + PRE + j, PRE + j)
    return 0

  lax.fori_loop(0, (NCHUNK - PRE) // 8, _oct, 0)
  scat_wait(NCHUNK - 1)

  plsc.subcore_barrier()

  @pl.when(sid < OUT_TILES)
  def _():
    pltpu.sync_copy(
        acc.at[pl.ds(sid * ROWS_PER_TILE, ROWS_PER_TILE), :],
        out_p.at[cid, pl.ds(sid * ROWS_PER_TILE, ROWS_PER_TILE), :],
    )
  if with_counts:
    @pl.when(sid < OUT_TILES)
    def _():
      for r in range(ROWS_PER_TILE // CNT_CHUNK):
        off = sid * ROWS_PER_TILE + r * CNT_CHUNK
        pltpu.sync_copy(cnt.at[pl.ds(off, CNT_CHUNK)],
                        zcnt.at[pl.ds(0, CNT_CHUNK)])
        pltpu.sync_copy(zcnt.at[pl.ds(0, CNT_CHUNK)],
                        out_c.at[pl.ds(cid * N_NODES + off, CNT_CHUNK)])


def _make_sc_agg(with_counts):
  mesh = plsc.VectorSubcoreMesh(
      core_axis_name="c", subcore_axis_name="s",
      num_cores=NUM_CORES, num_subcores=NUM_SUBCORES,
  )
  out_type = [jax.ShapeDtypeStruct((NUM_CORES, N_NODES, D), jnp.float32)]
  if with_counts:
    out_type.append(jax.ShapeDtypeStruct((NUM_CORES * N_NODES,), jnp.float32))
  scratch = []
  if with_counts:
    scratch += [
        pltpu.VMEM((CHUNK,), jnp.float32),
        pltpu.VMEM((CNT_PAD,), jnp.float32),
    ]
  scratch.append(pltpu.VMEM_SHARED((N_NODES, D), jnp.float32))
  if with_counts:
    scratch.append(pltpu.VMEM_SHARED((N_NODES,), jnp.float32))
  scratch += [pltpu.VMEM((CHUNK,), jnp.int32)] * NSRC
  scratch += [pltpu.VMEM((CHUNK,), jnp.int32)] * NBUF
  scratch += [pltpu.VMEM((CHUNK, D), jnp.float32)] * NBUF
  nsem = NSRC + 3 * NBUF + (NBUF if with_counts else 0)
  scratch += [pltpu.SemaphoreType.DMA] * nsem

  return pl.kernel(
      functools.partial(_sc_agg_body, with_counts),
      out_type=tuple(out_type) if with_counts else out_type[0],
      mesh=mesh,
      scratch_types=scratch,
  )


_sc_agg_with_counts = _make_sc_agg(True)
_sc_agg_no_counts = _make_sc_agg(False)

_DN_T = (((1,), (1,)), ((), ()))


def _tc_layer_body(relu, p_ref, c_ref, x_ref, wl_ref, wr_ref, bl_ref, o_ref):
  c = c_ref[0] + c_ref[1]
  inv = 1.0 / jnp.maximum(c, 1.0)
  mean = (p_ref[0] + p_ref[1]) * inv
  acc = lax.dot_general(mean, wl_ref[...], _DN_T,
                        preferred_element_type=jnp.float32)
  acc = acc + lax.dot_general(x_ref[...], wr_ref[...], _DN_T,
                              preferred_element_type=jnp.float32)
  acc = acc + bl_ref[...]
  if relu:
    acc = jnp.maximum(acc, 0.0)
  o_ref[...] = acc


def _make_tc_layer(relu, block=1000):
  nblk = N_NODES // block
  return pl.pallas_call(
      functools.partial(_tc_layer_body, relu),
      grid=(nblk,),
      in_specs=[
          pl.BlockSpec((NUM_CORES, block, D), lambda i: (0, i, 0)),
          pl.BlockSpec((NUM_CORES, block, 1), lambda i: (0, i, 0)),
          pl.BlockSpec((block, D), lambda i: (i, 0)),
          pl.BlockSpec((D, D), lambda i: (0, 0)),
          pl.BlockSpec((D, D), lambda i: (0, 0)),
          pl.BlockSpec((1, D), lambda i: (0, 0)),
      ],
      out_specs=pl.BlockSpec((block, D), lambda i: (i, 0)),
      out_shape=jax.ShapeDtypeStruct((N_NODES, D), jnp.float32),
  )


_tc_layer_relu = _make_tc_layer(True)
_tc_layer_lin = _make_tc_layer(False)


def kernel(x, edge_index, Wl1, bl1, Wr1, Wl2, bl2, Wr2):
  ei = edge_index.astype(jnp.int32).reshape(-1)

  p1, cnt = _sc_agg_with_counts(x, ei)
  cnt3 = cnt.reshape(NUM_CORES, N_NODES, 1)
  h = _tc_layer_relu(p1, cnt3, x, Wl1, Wr1, bl1.reshape(1, D))
  p2 = _sc_agg_no_counts(h, ei)
  out = _tc_layer_lin(p2, cnt3, h, Wl2, Wr2, bl2.reshape(1, D))
  return out


# speedup vs baseline: 15.2666x; 1.0144x over previous
import functools

import jax
import jax.numpy as jnp
from jax import lax
from jax.experimental import pallas as pl
from jax.experimental.pallas import tpu as pltpu
from jax.experimental.pallas import tpu_sc as plsc

N_NODES = 10000
N_EDGES = 320000
D = 128

NUM_CORES = 2
NUM_SUBCORES = 16
NW = NUM_CORES * NUM_SUBCORES
EDGES_PER_TILE = N_EDGES // NW
CHUNK = 80
NCHUNK = EDGES_PER_TILE // CHUNK
NBUF = 4
NSRC = 8
PRE = 5
OUT_TILES = 10
ROWS_PER_TILE = N_NODES // OUT_TILES
CNT_CHUNK = 200
CNT_PAD = 16 * ((CNT_CHUNK + 15) // 16)


def _sc_agg_body(with_counts, *refs):
  nsem = NSRC + 3 * NBUF + (NBUF if with_counts else 0)
  if with_counts:
    (x_hbm, ei_hbm, out_p, out_c, ones_v, zcnt, acc, cnt) = refs[:8]
    sv = refs[8:8 + NSRC]
    dv = refs[8 + NSRC:8 + NSRC + NBUF]
    rows = refs[8 + NSRC + NBUF:8 + NSRC + 2 * NBUF]
    sems = refs[8 + NSRC + 2 * NBUF:]
  else:
    (x_hbm, ei_hbm, out_p, acc) = refs[:4]
    ones_v = zcnt = cnt = None
    sv = refs[4:4 + NSRC]
    dv = refs[4 + NSRC:4 + NSRC + NBUF]
    rows = refs[4 + NSRC + NBUF:4 + NSRC + 2 * NBUF]
    sems = refs[4 + NSRC + 2 * NBUF:]
  isem = sems[:NSRC]
  jsem = sems[NSRC:NSRC + NBUF]
  gsem = sems[NSRC + NBUF:NSRC + 2 * NBUF]
  ssem = sems[NSRC + 2 * NBUF:NSRC + 3 * NBUF]
  csem = sems[NSRC + 3 * NBUF:] if with_counts else None

  cid = lax.axis_index("c")
  sid = lax.axis_index("s")
  wid = cid * NUM_SUBCORES + sid
  ebase = wid * EDGES_PER_TILE

  def ld_src(g, m):
    pltpu.async_copy(
        ei_hbm.at[pl.ds(ebase + g * CHUNK, CHUNK)], sv[m % NSRC],
        isem[m % NSRC])

  def src_wait(m):
    pltpu.make_async_copy(
        ei_hbm.at[pl.ds(0, CHUNK)], sv[m % NSRC], isem[m % NSRC]).wait()

  def ld_dst(g, m):
    pltpu.async_copy(
        ei_hbm.at[pl.ds(N_EDGES + ebase + g * CHUNK, CHUNK)], dv[m % NBUF],
        jsem[m % NBUF])

  def dst_wait(m):
    pltpu.make_async_copy(
        ei_hbm.at[pl.ds(0, CHUNK)], dv[m % NBUF], jsem[m % NBUF]).wait()

  def gat(m):
    pltpu.async_copy(x_hbm.at[sv[m % NSRC]], rows[m % NBUF], gsem[m % NBUF])

  def gat_wait(m):
    pltpu.make_async_copy(
        x_hbm.at[sv[m % NSRC]], rows[m % NBUF], gsem[m % NBUF]).wait()

  def scat(m):
    pltpu.async_copy(rows[m % NBUF], acc.at[dv[m % NBUF]], ssem[m % NBUF],
                     add=True)
    if with_counts:
      pltpu.async_copy(ones_v, cnt.at[dv[m % NBUF]], csem[m % NBUF], add=True)

  def scat_wait(m):
    pltpu.make_async_copy(
        rows[m % NBUF], acc.at[dv[m % NBUF]], ssem[m % NBUF]).wait()
    if with_counts:
      pltpu.make_async_copy(
          ones_v, cnt.at[dv[m % NBUF]], csem[m % NBUF]).wait()

  def step(g, m, first=False):
    gat_wait(m)
    dst_wait(m)
    scat(m)
    if not first:
      scat_wait(m - 1)

    @pl.when(g + 4 < NCHUNK)
    def _():
      ld_src(g + 4, m + 4)

    @pl.when(g + 3 < NCHUNK)
    def _():
      ld_dst(g + 3, m + 3)
      src_wait(m + 3)
      gat(m + 3)

  for g in range(4):
    ld_src(g, g)
  for g in range(3):
    ld_dst(g, g)
  for g in range(3):
    src_wait(g)
    gat(g)

  zbuf = rows[NBUF - 1]
  zeros16 = jnp.zeros((16,), jnp.float32)

  def _zrow(i, _):
    for j in range(D // 16):
      zbuf[i, pl.ds(j * 16, 16)] = zeros16
    return 0

  lax.fori_loop(0, CHUNK, _zrow, 0)

  @pl.when(sid < OUT_TILES)
  def _():
    for r in range(ROWS_PER_TILE // CHUNK):
      pltpu.sync_copy(
          zbuf, acc.at[pl.ds(sid * ROWS_PER_TILE + r * CHUNK, CHUNK), :])
    pltpu.sync_copy(
        zbuf.at[pl.ds(0, ROWS_PER_TILE % CHUNK), :],
        acc.at[pl.ds(sid * ROWS_PER_TILE + ROWS_PER_TILE - ROWS_PER_TILE % CHUNK,
                     ROWS_PER_TILE % CHUNK), :])

  if with_counts:
    ones16 = jnp.ones((16,), jnp.float32)
    for j in range(CHUNK // 16):
      ones_v[pl.ds(j * 16, 16)] = ones16

    def _zc(i, _):
      zcnt[pl.ds(i * 16, 16)] = zeros16
      return 0

    lax.fori_loop(0, CNT_PAD // 16, _zc, 0)

    @pl.when(sid < OUT_TILES)
    def _():
      for r in range(ROWS_PER_TILE // CNT_CHUNK):
        pltpu.sync_copy(
            zcnt.at[pl.ds(0, CNT_CHUNK)],
            cnt.at[pl.ds(sid * ROWS_PER_TILE + r * CNT_CHUNK, CNT_CHUNK)])

  plsc.subcore_barrier()

  step(0, 0, first=True)
  for g in range(1, PRE):
    step(g, g)

  def _oct(k, _):
    for j in range(8):
      step(8 * k + PRE + j, PRE + j)
    return 0

  lax.fori_loop(0, (NCHUNK - PRE) // 8, _oct, 0)
  scat_wait(NCHUNK - 1)

  plsc.subcore_barrier()

  @pl.when(sid < OUT_TILES)
  def _():
    pltpu.sync_copy(
        acc.at[pl.ds(sid * ROWS_PER_TILE, ROWS_PER_TILE), :],
        out_p.at[cid, pl.ds(sid * ROWS_PER_TILE, ROWS_PER_TILE), :],
    )
  if with_counts:
    @pl.when(sid < OUT_TILES)
    def _():
      for r in range(ROWS_PER_TILE // CNT_CHUNK):
        off = sid * ROWS_PER_TILE + r * CNT_CHUNK
        pltpu.sync_copy(cnt.at[pl.ds(off, CNT_CHUNK)],
                        zcnt.at[pl.ds(0, CNT_CHUNK)])
        pltpu.sync_copy(zcnt.at[pl.ds(0, CNT_CHUNK)],
                        out_c.at[pl.ds(cid * N_NODES + off, CNT_CHUNK)])


def _make_sc_agg(with_counts):
  mesh = plsc.VectorSubcoreMesh(
      core_axis_name="c", subcore_axis_name="s",
      num_cores=NUM_CORES, num_subcores=NUM_SUBCORES,
  )
  out_type = [jax.ShapeDtypeStruct((NUM_CORES, N_NODES, D), jnp.float32)]
  if with_counts:
    out_type.append(jax.ShapeDtypeStruct((NUM_CORES * N_NODES,), jnp.float32))
  scratch = []
  if with_counts:
    scratch += [
        pltpu.VMEM((CHUNK,), jnp.float32),
        pltpu.VMEM((CNT_PAD,), jnp.float32),
    ]
  scratch.append(pltpu.VMEM_SHARED((N_NODES, D), jnp.float32))
  if with_counts:
    scratch.append(pltpu.VMEM_SHARED((N_NODES,), jnp.float32))
  scratch += [pltpu.VMEM((CHUNK,), jnp.int32)] * NSRC
  scratch += [pltpu.VMEM((CHUNK,), jnp.int32)] * NBUF
  scratch += [pltpu.VMEM((CHUNK, D), jnp.float32)] * NBUF
  nsem = NSRC + 3 * NBUF + (NBUF if with_counts else 0)
  scratch += [pltpu.SemaphoreType.DMA] * nsem

  return pl.kernel(
      functools.partial(_sc_agg_body, with_counts),
      out_type=tuple(out_type) if with_counts else out_type[0],
      mesh=mesh,
      scratch_types=scratch,
  )


_sc_agg_with_counts = _make_sc_agg(True)
_sc_agg_no_counts = _make_sc_agg(False)

_DN_T = (((1,), (1,)), ((), ()))


def _tc_layer_body(relu, p_ref, c_ref, x_ref, wl_ref, wr_ref, bl_ref, o_ref):
  c = c_ref[0] + c_ref[1]
  inv = 1.0 / jnp.maximum(c, 1.0)
  mean = (p_ref[0] + p_ref[1]) * inv
  acc = lax.dot_general(mean, wl_ref[...], _DN_T,
                        preferred_element_type=jnp.float32)
  acc = acc + lax.dot_general(x_ref[...], wr_ref[...], _DN_T,
                              preferred_element_type=jnp.float32)
  acc = acc + bl_ref[...]
  if relu:
    acc = jnp.maximum(acc, 0.0)
  o_ref[...] = acc


def _make_tc_layer(relu, block=1000):
  nblk = N_NODES // block
  return pl.pallas_call(
      functools.partial(_tc_layer_body, relu),
      grid=(nblk,),
      in_specs=[
          pl.BlockSpec((NUM_CORES, block, D), lambda i: (0, i, 0)),
          pl.BlockSpec((NUM_CORES, block, 1), lambda i: (0, i, 0)),
          pl.BlockSpec((block, D), lambda i: (i, 0)),
          pl.BlockSpec((D, D), lambda i: (0, 0)),
          pl.BlockSpec((D, D), lambda i: (0, 0)),
          pl.BlockSpec((1, D), lambda i: (0, 0)),
      ],
      out_specs=pl.BlockSpec((block, D), lambda i: (i, 0)),
      out_shape=jax.ShapeDtypeStruct((N_NODES, D), jnp.float32),
  )


_tc_layer_relu = _make_tc_layer(True)
_tc_layer_lin = _make_tc_layer(False)


def kernel(x, edge_index, Wl1, bl1, Wr1, Wl2, bl2, Wr2):
  ei = edge_index.astype(jnp.int32).reshape(-1)

  p1, cnt = _sc_agg_with_counts(x, ei)
  cnt3 = cnt.reshape(NUM_CORES, N_NODES, 1)
  h = _tc_layer_relu(p1, cnt3, x, Wl1, Wr1, bl1.reshape(1, D))
  p2 = _sc_agg_no_counts(h, ei)
  out = _tc_layer_lin(p2, cnt3, h, Wl2, Wr2, bl2.reshape(1, D))
  return out


# speedup vs baseline: 15.3005x; 1.0022x over previous
import functools

import jax
import jax.numpy as jnp
from jax import lax
from jax.experimental import pallas as pl
from jax.experimental.pallas import tpu as pltpu
from jax.experimental.pallas import tpu_sc as plsc

N_NODES = 10000
N_EDGES = 320000
D = 128

NUM_CORES = 2
NUM_SUBCORES = 16
NW = NUM_CORES * NUM_SUBCORES
EDGES_PER_TILE = N_EDGES // NW
CHUNK = 80
NCHUNK = EDGES_PER_TILE // CHUNK
NBUF = 4
NSRC = 8
PRE = 5
OUT_TILES = 10
ROWS_PER_TILE = N_NODES // OUT_TILES
CNT_CHUNK = 200
CNT_PAD = 16 * ((CNT_CHUNK + 15) // 16)


def _sc_agg_body(with_counts, *refs):
  nsem = NSRC + 3 * NBUF + (NBUF if with_counts else 0)
  if with_counts:
    (x_hbm, ei_hbm, out_p, out_c, ones_v, zcnt, acc, cnt) = refs[:8]
    sv = refs[8:8 + NSRC]
    dv = refs[8 + NSRC:8 + NSRC + NBUF]
    rows = refs[8 + NSRC + NBUF:8 + NSRC + 2 * NBUF]
    sems = refs[8 + NSRC + 2 * NBUF:]
  else:
    (x_hbm, ei_hbm, out_p, acc) = refs[:4]
    ones_v = zcnt = cnt = None
    sv = refs[4:4 + NSRC]
    dv = refs[4 + NSRC:4 + NSRC + NBUF]
    rows = refs[4 + NSRC + NBUF:4 + NSRC + 2 * NBUF]
    sems = refs[4 + NSRC + 2 * NBUF:]
  isem = sems[:NSRC]
  jsem = sems[NSRC:NSRC + NBUF]
  gsem = sems[NSRC + NBUF:NSRC + 2 * NBUF]
  ssem = sems[NSRC + 2 * NBUF:NSRC + 3 * NBUF]
  csem = sems[NSRC + 3 * NBUF:] if with_counts else None

  cid = lax.axis_index("c")
  sid = lax.axis_index("s")
  wid = cid * NUM_SUBCORES + sid
  ebase = wid * EDGES_PER_TILE

  def ld_src(g, m):
    pltpu.async_copy(
        ei_hbm.at[pl.ds(ebase + g * CHUNK, CHUNK)], sv[m % NSRC],
        isem[m % NSRC])

  def src_wait(m):
    pltpu.make_async_copy(
        ei_hbm.at[pl.ds(0, CHUNK)], sv[m % NSRC], isem[m % NSRC]).wait()

  def ld_dst(g, m):
    pltpu.async_copy(
        ei_hbm.at[pl.ds(N_EDGES + ebase + g * CHUNK, CHUNK)], dv[m % NBUF],
        jsem[m % NBUF])

  def dst_wait(m):
    pltpu.make_async_copy(
        ei_hbm.at[pl.ds(0, CHUNK)], dv[m % NBUF], jsem[m % NBUF]).wait()

  def gat(m):
    pltpu.async_copy(x_hbm.at[sv[m % NSRC]], rows[m % NBUF], gsem[m % NBUF])

  def gat_wait(m):
    pltpu.make_async_copy(
        x_hbm.at[sv[m % NSRC]], rows[m % NBUF], gsem[m % NBUF]).wait()

  def scat(m):
    pltpu.async_copy(rows[m % NBUF], acc.at[dv[m % NBUF]], ssem[m % NBUF],
                     add=True)
    if with_counts:
      pltpu.async_copy(ones_v, cnt.at[dv[m % NBUF]], csem[m % NBUF], add=True)

  def scat_wait(m):
    pltpu.make_async_copy(
        rows[m % NBUF], acc.at[dv[m % NBUF]], ssem[m % NBUF]).wait()
    if with_counts:
      pltpu.make_async_copy(
          ones_v, cnt.at[dv[m % NBUF]], csem[m % NBUF]).wait()

  def step(g, m, first=False):
    gat_wait(m)
    dst_wait(m)
    scat(m)
    if not first:
      scat_wait(m - 1)

    @pl.when(g + 4 < NCHUNK)
    def _():
      ld_src(g + 4, m + 4)

    @pl.when(g + 3 < NCHUNK)
    def _():
      ld_dst(g + 3, m + 3)
      src_wait(m + 3)
      gat(m + 3)

  for g in range(4):
    ld_src(g, g)
  for g in range(3):
    ld_dst(g, g)
  for g in range(3):
    src_wait(g)
    gat(g)

  zbuf = rows[NBUF - 1]
  zeros16 = jnp.zeros((16,), jnp.float32)

  def _zrow(i, _):
    for j in range(D // 16):
      zbuf[i, pl.ds(j * 16, 16)] = zeros16
    return 0

  lax.fori_loop(0, CHUNK, _zrow, 0)

  @pl.when(sid < OUT_TILES)
  def _():
    for r in range(ROWS_PER_TILE // CHUNK):
      pltpu.sync_copy(
          zbuf, acc.at[pl.ds(sid * ROWS_PER_TILE + r * CHUNK, CHUNK), :])
    pltpu.sync_copy(
        zbuf.at[pl.ds(0, ROWS_PER_TILE % CHUNK), :],
        acc.at[pl.ds(sid * ROWS_PER_TILE + ROWS_PER_TILE - ROWS_PER_TILE % CHUNK,
                     ROWS_PER_TILE % CHUNK), :])

  if with_counts:
    ones16 = jnp.ones((16,), jnp.float32)
    for j in range(CHUNK // 16):
      ones_v[pl.ds(j * 16, 16)] = ones16

    def _zc(i, _):
      zcnt[pl.ds(i * 16, 16)] = zeros16
      return 0

    lax.fori_loop(0, CNT_PAD // 16, _zc, 0)

    @pl.when(sid < OUT_TILES)
    def _():
      for r in range(ROWS_PER_TILE // CNT_CHUNK):
        pltpu.sync_copy(
            zcnt.at[pl.ds(0, CNT_CHUNK)],
            cnt.at[pl.ds(sid * ROWS_PER_TILE + r * CNT_CHUNK, CNT_CHUNK)])

  plsc.subcore_barrier()

  step(0, 0, first=True)
  for g in range(1, PRE):
    step(g, g)

  def _oct(k, _):
    for j in range(8):
      step(8 * k + PRE + j, PRE + j)
    return 0

  lax.fori_loop(0, (NCHUNK - PRE) // 8, _oct, 0)
  scat_wait(NCHUNK - 1)

  plsc.subcore_barrier()

  @pl.when(sid < OUT_TILES)
  def _():
    pltpu.sync_copy(
        acc.at[pl.ds(sid * ROWS_PER_TILE, ROWS_PER_TILE), :],
        out_p.at[cid, pl.ds(sid * ROWS_PER_TILE, ROWS_PER_TILE), :],
    )
  if with_counts:
    @pl.when(sid < OUT_TILES)
    def _():
      for r in range(ROWS_PER_TILE // CNT_CHUNK):
        off = sid * ROWS_PER_TILE + r * CNT_CHUNK
        pltpu.sync_copy(cnt.at[pl.ds(off, CNT_CHUNK)],
                        zcnt.at[pl.ds(0, CNT_CHUNK)])
        pltpu.sync_copy(zcnt.at[pl.ds(0, CNT_CHUNK)],
                        out_c.at[pl.ds(cid * N_NODES + off, CNT_CHUNK)])


def _make_sc_agg(with_counts):
  mesh = plsc.VectorSubcoreMesh(
      core_axis_name="c", subcore_axis_name="s",
      num_cores=NUM_CORES, num_subcores=NUM_SUBCORES,
  )
  out_type = [jax.ShapeDtypeStruct((NUM_CORES, N_NODES, D), jnp.float32)]
  if with_counts:
    out_type.append(jax.ShapeDtypeStruct((NUM_CORES * N_NODES,), jnp.float32))
  scratch = []
  if with_counts:
    scratch += [
        pltpu.VMEM((CHUNK,), jnp.float32),
        pltpu.VMEM((CNT_PAD,), jnp.float32),
    ]
  scratch.append(pltpu.VMEM_SHARED((N_NODES, D), jnp.float32))
  if with_counts:
    scratch.append(pltpu.VMEM_SHARED((N_NODES,), jnp.float32))
  scratch += [pltpu.VMEM((CHUNK,), jnp.int32)] * NSRC
  scratch += [pltpu.VMEM((CHUNK,), jnp.int32)] * NBUF
  scratch += [pltpu.VMEM((CHUNK, D), jnp.float32)] * NBUF
  nsem = NSRC + 3 * NBUF + (NBUF if with_counts else 0)
  scratch += [pltpu.SemaphoreType.DMA] * nsem

  return pl.kernel(
      functools.partial(_sc_agg_body, with_counts),
      out_type=tuple(out_type) if with_counts else out_type[0],
      mesh=mesh,
      scratch_types=scratch,
  )


_sc_agg_with_counts = _make_sc_agg(True)
_sc_agg_no_counts = _make_sc_agg(False)

_DN_T = (((1,), (1,)), ((), ()))


def _tc_mm_body(x_ref, w_ref, o_ref):
  o_ref[...] = lax.dot_general(x_ref[...], w_ref[...], _DN_T,
                               preferred_element_type=jnp.float32)


def _tc_combine_body(relu, p_ref, c_ref, xr_ref, wl_ref, bl_ref, o_ref):
  c = c_ref[0] + c_ref[1]
  inv = 1.0 / jnp.maximum(c, 1.0)
  mean = (p_ref[0] + p_ref[1]) * inv
  acc = lax.dot_general(mean, wl_ref[...], _DN_T,
                        preferred_element_type=jnp.float32)
  acc = acc + xr_ref[...] + bl_ref[...]
  if relu:
    acc = jnp.maximum(acc, 0.0)
  o_ref[...] = acc


_BLOCK = 1000
_NBLK = N_NODES // _BLOCK

_tc_mm = pl.pallas_call(
    _tc_mm_body,
    grid=(_NBLK,),
    in_specs=[
        pl.BlockSpec((_BLOCK, D), lambda i: (i, 0)),
        pl.BlockSpec((D, D), lambda i: (0, 0)),
    ],
    out_specs=pl.BlockSpec((_BLOCK, D), lambda i: (i, 0)),
    out_shape=jax.ShapeDtypeStruct((N_NODES, D), jnp.float32),
)


def _make_tc_combine(relu):
  return pl.pallas_call(
      functools.partial(_tc_combine_body, relu),
      grid=(_NBLK,),
      in_specs=[
          pl.BlockSpec((NUM_CORES, _BLOCK, D), lambda i: (0, i, 0)),
          pl.BlockSpec((NUM_CORES, _BLOCK, 1), lambda i: (0, i, 0)),
          pl.BlockSpec((_BLOCK, D), lambda i: (i, 0)),
          pl.BlockSpec((D, D), lambda i: (0, 0)),
          pl.BlockSpec((1, D), lambda i: (0, 0)),
      ],
      out_specs=pl.BlockSpec((_BLOCK, D), lambda i: (i, 0)),
      out_shape=jax.ShapeDtypeStruct((N_NODES, D), jnp.float32),
  )


_tc_combine_relu = _make_tc_combine(True)
_tc_combine_lin = _make_tc_combine(False)


def kernel(x, edge_index, Wl1, bl1, Wr1, Wl2, bl2, Wr2):
  ei = edge_index.astype(jnp.int32).reshape(-1)

  xr1 = _tc_mm(x, Wr1)
  p1, cnt = _sc_agg_with_counts(x, ei)
  cnt3 = cnt.reshape(NUM_CORES, N_NODES, 1)
  h = _tc_combine_relu(p1, cnt3, xr1, Wl1, bl1.reshape(1, D))
  xr2 = _tc_mm(h, Wr2)
  p2 = _sc_agg_no_counts(h, ei)
  out = _tc_combine_lin(p2, cnt3, xr2, Wl2, bl2.reshape(1, D))
  return out
